# baseline (speedup 1.0000x reference)
import jax
import jax.numpy as jnp
from jax.experimental import pallas as pl
from jax.experimental.pallas import tpu as pltpu

BATCH = 32
SEQ_LEN = 8192
N_ALLELES = 4
D_MODEL = 64
GROUP = 128 // N_ALLELES
WIDE = GROUP * D_MODEL
ROWS = BATCH * SEQ_LEN // GROUP
ROWS_PER_B = ROWS // BATCH
W_TILE = 128
S_TILES = ROWS_PER_B // W_TILE


def _body(x_ref, ap_ref, p_ref, o_ref, w_ref):
    @pl.when((pl.program_id(0) == 0) & (pl.program_id(1) == 0))
    def _init():
        j = jax.lax.broadcasted_iota(jnp.int32, (128, WIDE), 0)
        c = jax.lax.broadcasted_iota(jnp.int32, (128, WIDE), 1)
        mask = (j >> 2) == (c >> 6)
        w_ref[...] = jnp.where(mask, ap_ref[...], jnp.bfloat16(0))

    emb = jax.lax.dot_general(
        x_ref[...].astype(jnp.bfloat16), w_ref[...],
        dimension_numbers=(((1,), (0,)), ((), ())),
        preferred_element_type=jnp.float32,
    )
    o_ref[...] = emb + p_ref[...]


def kernel(x, allele_embedding, position_table):
    xw = x.reshape(ROWS, 128)
    a_pad = jnp.tile(allele_embedding.astype(jnp.bfloat16), (GROUP, GROUP))
    pw = position_table.reshape(-1, WIDE)
    out = pl.pallas_call(
        _body,
        grid=(S_TILES, BATCH),
        in_specs=[
            pl.BlockSpec((W_TILE, 128), lambda s, b: (b * S_TILES + s, 0)),
            pl.BlockSpec((128, WIDE), lambda s, b: (0, 0)),
            pl.BlockSpec((W_TILE, WIDE), lambda s, b: (s, 0)),
        ],
        out_specs=pl.BlockSpec((W_TILE, WIDE), lambda s, b: (b * S_TILES + s, 0)),
        out_shape=jax.ShapeDtypeStruct((ROWS, WIDE), jnp.float32),
        scratch_shapes=[pltpu.VMEM((128, WIDE), jnp.bfloat16)],
    )(xw, a_pad, pw)
    return out.reshape(BATCH, SEQ_LEN, D_MODEL)


# speedup vs baseline: 1.3383x; 1.3383x over previous
import jax
import jax.numpy as jnp
from jax.experimental import pallas as pl

BATCH = 32
SEQ_LEN = 8192
N_ALLELES = 4
D_MODEL = 64
S_TILE = 2048
S_TILES = SEQ_LEN // S_TILE


def _body(x_ref, a_ref, p_ref, o_ref):
    emb = jax.lax.dot_general(
        x_ref[0], a_ref[...],
        dimension_numbers=(((1,), (0,)), ((), ())),
        preferred_element_type=jnp.float32,
    )
    o_ref[0] = emb + p_ref[...]


def kernel(x, allele_embedding, position_table):
    return pl.pallas_call(
        _body,
        grid=(S_TILES, BATCH),
        in_specs=[
            pl.BlockSpec((1, S_TILE, N_ALLELES), lambda s, b: (b, s, 0)),
            pl.BlockSpec((N_ALLELES, D_MODEL), lambda s, b: (0, 0)),
            pl.BlockSpec((S_TILE, D_MODEL), lambda s, b: (s, 0)),
        ],
        out_specs=pl.BlockSpec((1, S_TILE, D_MODEL), lambda s, b: (b, s, 0)),
        out_shape=jax.ShapeDtypeStruct((BATCH, SEQ_LEN, D_MODEL), jnp.float32),
    )(x, allele_embedding, position_table)
